# Initial kernel scaffold; baseline (speedup 1.0000x reference)
import jax
import jax.numpy as jnp
from jax.experimental import pallas as pl


def kernel(x, edge_index, batch, conv_W0, conv_b0, conv_W1, conv_b1, conv_W2, conv_b2, lin1_W, lin1_b, lin2_W, lin2_b):
    raise NotImplementedError("write your pallas kernel here")


# speedup vs baseline: 19.8204x; 19.8204x over previous
import functools

import jax
import jax.numpy as jnp
from jax import lax
from jax.experimental import pallas as pl
from jax.experimental.pallas import tpu as pltpu
from jax.experimental.pallas import tpu_sc as plsc

N = 10000
D = 128
E = 320000
G = 64
H = 256

NC = 2
NS = 16
NW = NC * NS
CHUNK = 125
CPT = E // (NW * CHUNK)
EPT = E // NW
NPT = 624
NPT_LAST = N - (NS - 1) * NPT
NPAD = 10240
DPT = NPAD // NS

BLK = 1000
NBLK = N // BLK


def _mesh():
    return plsc.VectorSubcoreMesh(core_axis_name="c", subcore_axis_name="s")


def _sc_deg_body(dst_hbm, zeros_hbm, out_hbm, dstbuf, ones_v, sem, deg_sh):
    c = lax.axis_index("c")
    s = lax.axis_index("s")
    wid = c * NS + s
    pltpu.sync_copy(dst_hbm.at[pl.ds(wid * CPT, CPT)], dstbuf)
    for k in range(CHUNK // 16):
        ones_v[pl.ds(16 * k, 16)] = jnp.ones((16,), jnp.float32)
    if CHUNK % 16:
        ones_v[pl.ds(CHUNK - 16, 16)] = jnp.ones((16,), jnp.float32)
    pltpu.sync_copy(zeros_hbm.at[pl.ds(s * DPT, DPT)], deg_sh.at[pl.ds(s * DPT, DPT)])
    plsc.subcore_barrier()

    def body(j, carry):
        pltpu.sync_copy(ones_v, deg_sh.at[dstbuf.at[j]], add=True)
        return carry

    lax.fori_loop(0, CPT, body, 0)
    plsc.subcore_barrier()
    pltpu.sync_copy(deg_sh.at[pl.ds(s * DPT, DPT)], out_hbm.at[c, pl.ds(s * DPT, DPT)])


def _sc_degree(dst2, zeros1d):
    fn = pl.kernel(
        _sc_deg_body,
        out_type=jax.ShapeDtypeStruct((NC, NPAD), jnp.float32),
        mesh=_mesh(),
        scratch_types=[
            pltpu.VMEM((CPT, CHUNK), jnp.int32),
            pltpu.VMEM((CHUNK,), jnp.float32),
            pltpu.SemaphoreType.DMA,
            pltpu.VMEM_SHARED((NPAD,), jnp.float32),
        ],
    )
    return fn(dst2, zeros1d)


def _sc_edge_body(g_hbm, src_hbm, dst_hbm, zeros_hbm, out_hbm,
                  srcbuf, dstbuf, rowbuf, sem, acc_sh):
    c = lax.axis_index("c")
    s = lax.axis_index("s")
    wid = c * NS + s
    node_base = s * NPT
    pltpu.sync_copy(src_hbm.at[pl.ds(wid * CPT, CPT)], srcbuf)
    pltpu.sync_copy(dst_hbm.at[pl.ds(wid * CPT, CPT)], dstbuf)

    @pl.when(jnp.logical_and(c == 0, s < NS - 1))
    def _():
        pltpu.sync_copy(g_hbm.at[pl.ds(node_base, NPT)],
                        acc_sh.at[pl.ds(node_base, NPT)])

    @pl.when(jnp.logical_and(c == 0, s == NS - 1))
    def _():
        pltpu.sync_copy(g_hbm.at[pl.ds(node_base, NPT_LAST)],
                        acc_sh.at[pl.ds(node_base, NPT_LAST)])

    @pl.when(jnp.logical_and(c != 0, s < NS - 1))
    def _():
        pltpu.sync_copy(zeros_hbm.at[pl.ds(node_base, NPT)],
                        acc_sh.at[pl.ds(node_base, NPT)])

    @pl.when(jnp.logical_and(c != 0, s == NS - 1))
    def _():
        pltpu.sync_copy(zeros_hbm.at[pl.ds(node_base, NPT_LAST)],
                        acc_sh.at[pl.ds(node_base, NPT_LAST)])

    plsc.subcore_barrier()

    def body(j, carry):
        pltpu.async_copy(g_hbm.at[srcbuf.at[j]], rowbuf, sem).wait()
        pltpu.sync_copy(rowbuf, acc_sh.at[dstbuf.at[j]], add=True)
        return carry

    lax.fori_loop(0, CPT, body, 0)
    plsc.subcore_barrier()

    @pl.when(s < NS - 1)
    def _():
        pltpu.sync_copy(acc_sh.at[pl.ds(node_base, NPT)],
                        out_hbm.at[c, pl.ds(node_base, NPT)])

    @pl.when(s == NS - 1)
    def _():
        pltpu.sync_copy(acc_sh.at[pl.ds(node_base, NPT_LAST)],
                        out_hbm.at[c, pl.ds(node_base, NPT_LAST)])


def _sc_edge(g, src2, dst2, zeros2d):
    fn = pl.kernel(
        _sc_edge_body,
        out_type=jax.ShapeDtypeStruct((NC, N, D), jnp.float32),
        mesh=_mesh(),
        scratch_types=[
            pltpu.VMEM((CPT, CHUNK), jnp.int32),
            pltpu.VMEM((CPT, CHUNK), jnp.int32),
            pltpu.VMEM((CHUNK, D), jnp.float32),
            pltpu.SemaphoreType.DMA,
            pltpu.VMEM_SHARED((N, D), jnp.float32),
        ],
    )
    return fn(g, src2, dst2, zeros2d)


def _tc_first_body(x_ref, w_ref, d0_ref, d1_ref, g_ref, dinv_ref):
    deg = d0_ref[0, 0, :] + d1_ref[0, 0, :] + 1.0
    dinv = lax.rsqrt(deg)
    dinv_ref[0, 0, :] = dinv
    xw = jnp.dot(x_ref[...], w_ref[...], preferred_element_type=jnp.float32)
    g_ref[...] = xw * dinv[:, None]


def _tc_mid_body(a_ref, dinv_ref, b_ref, w_ref, g_ref):
    dinv = dinv_ref[0, 0, :]
    acc = a_ref[0, :, :] + a_ref[1, :, :]
    h = jnp.maximum(acc * dinv[:, None] + b_ref[...], 0.0)
    hw = jnp.dot(h, w_ref[...], preferred_element_type=jnp.float32)
    g_ref[...] = hw * dinv[:, None]


def _tc_final_body(a_ref, dinv_ref, b_ref, batch_ref, w1_ref, b1_ref,
                   w2_ref, b2_ref, out_ref, psum, cnt):
    i = pl.program_id(0)

    @pl.when(i == 0)
    def _():
        psum[...] = jnp.zeros_like(psum)
        cnt[...] = jnp.zeros_like(cnt)

    dinv = dinv_ref[0, 0, :]
    h = jnp.maximum((a_ref[0, :, :] + a_ref[1, :, :]) * dinv[:, None]
                    + b_ref[...], 0.0)
    bt = batch_ref[0, 0, :]
    seg = lax.broadcasted_iota(jnp.int32, (G, BLK), 0)
    onehot = (bt[None, :] == seg).astype(jnp.float32)
    psum[...] += jnp.dot(onehot, h, preferred_element_type=jnp.float32)
    cnt[...] += jnp.broadcast_to(
        jnp.sum(onehot, axis=1, keepdims=True), cnt.shape)

    @pl.when(i == pl.num_programs(0) - 1)
    def _():
        pooled = psum[...] / jnp.maximum(cnt[...], 1.0)
        z = jnp.maximum(
            jnp.dot(pooled, w1_ref[...], preferred_element_type=jnp.float32)
            + b1_ref[...], 0.0)
        out_ref[...] = (jnp.sum(z * w2_ref[...], axis=1, keepdims=True)
                        + b2_ref[...])


def _tc_first(x, W0, deg0, deg1):
    return pl.pallas_call(
        _tc_first_body,
        grid=(NBLK,),
        in_specs=[
            pl.BlockSpec((BLK, D), lambda i: (i, 0)),
            pl.BlockSpec((D, D), lambda i: (0, 0)),
            pl.BlockSpec((1, 1, BLK), lambda i: (i, 0, 0)),
            pl.BlockSpec((1, 1, BLK), lambda i: (i, 0, 0)),
        ],
        out_specs=[
            pl.BlockSpec((BLK, D), lambda i: (i, 0)),
            pl.BlockSpec((1, 1, BLK), lambda i: (i, 0, 0)),
        ],
        out_shape=[
            jax.ShapeDtypeStruct((N, D), jnp.float32),
            jax.ShapeDtypeStruct((NBLK, 1, BLK), jnp.float32),
        ],
    )(x, W0, deg0, deg1)


def _tc_mid(acc, dinv3, b_row, W):
    return pl.pallas_call(
        _tc_mid_body,
        grid=(NBLK,),
        in_specs=[
            pl.BlockSpec((NC, BLK, D), lambda i: (0, i, 0)),
            pl.BlockSpec((1, 1, BLK), lambda i: (i, 0, 0)),
            pl.BlockSpec((1, D), lambda i: (0, 0)),
            pl.BlockSpec((D, D), lambda i: (0, 0)),
        ],
        out_specs=pl.BlockSpec((BLK, D), lambda i: (i, 0)),
        out_shape=jax.ShapeDtypeStruct((N, D), jnp.float32),
    )(acc, dinv3, b_row, W)


def _tc_final(acc, dinv3, b_row, batch3, W1, b1_row, w2_row, b2_11):
    return pl.pallas_call(
        _tc_final_body,
        grid=(NBLK,),
        in_specs=[
            pl.BlockSpec((NC, BLK, D), lambda i: (0, i, 0)),
            pl.BlockSpec((1, 1, BLK), lambda i: (i, 0, 0)),
            pl.BlockSpec((1, D), lambda i: (0, 0)),
            pl.BlockSpec((1, 1, BLK), lambda i: (i, 0, 0)),
            pl.BlockSpec((D, H), lambda i: (0, 0)),
            pl.BlockSpec((1, H), lambda i: (0, 0)),
            pl.BlockSpec((1, H), lambda i: (0, 0)),
            pl.BlockSpec((1, 1), lambda i: (0, 0)),
        ],
        out_specs=pl.BlockSpec((G, 1), lambda i: (0, 0)),
        out_shape=jax.ShapeDtypeStruct((G, 1), jnp.float32),
        scratch_shapes=[
            pltpu.VMEM((G, D), jnp.float32),
            pltpu.VMEM((G, D), jnp.float32),
        ],
    )(acc, dinv3, b_row, batch3, W1, b1_row, w2_row, b2_11)


def kernel(x, edge_index, batch, conv_W0, conv_b0, conv_W1, conv_b1,
           conv_W2, conv_b2, lin1_W, lin1_b, lin2_W, lin2_b):
    src2 = edge_index[0].reshape(E // CHUNK, CHUNK)
    dst2 = edge_index[1].reshape(E // CHUNK, CHUNK)
    zeros2d = jnp.zeros((N, D), jnp.float32)
    zeros1d = jnp.zeros((NPAD,), jnp.float32)
    batch3 = batch.reshape(NBLK, 1, BLK)

    deg_pair = _sc_degree(dst2, zeros1d)
    deg0 = deg_pair[0, :N].reshape(NBLK, 1, BLK)
    deg1 = deg_pair[1, :N].reshape(NBLK, 1, BLK)

    g, dinv3 = _tc_first(x, conv_W0, deg0, deg1)

    bs = [conv_b0, conv_b1, conv_b2]
    Ws = [conv_W1, conv_W2]
    for l in range(2):
        acc = _sc_edge(g, src2, dst2, zeros2d)
        g = _tc_mid(acc, dinv3, bs[l].reshape(1, D), Ws[l])
    acc = _sc_edge(g, src2, dst2, zeros2d)

    return _tc_final(acc, dinv3, bs[2].reshape(1, D), batch3,
                     lin1_W, lin1_b.reshape(1, H), lin2_W.reshape(1, H),
                     lin2_b.reshape(1, 1))


# speedup vs baseline: 29.1766x; 1.4721x over previous
import functools

import jax
import jax.numpy as jnp
from jax import lax
from jax.experimental import pallas as pl
from jax.experimental.pallas import tpu as pltpu
from jax.experimental.pallas import tpu_sc as plsc

N = 10000
D = 128
E = 320000
G = 64
H = 256

NC = 2
NS = 16
NW = NC * NS
CHUNK = 125
CPT = E // (NW * CHUNK)
EPT = E // NW
NPT = 624
NPT_LAST = N - (NS - 1) * NPT
HCPT = CPT // 2
NPAD = 10240
DPT = NPAD // NS

BLK = 1000
NBLK = N // BLK


def _mesh():
    return plsc.VectorSubcoreMesh(core_axis_name="c", subcore_axis_name="s")


def _sc_deg_body(dst_hbm, zeros_hbm, out_hbm, dstbuf, ones_v, sem, deg_sh):
    c = lax.axis_index("c")
    s = lax.axis_index("s")
    wid = c * NS + s
    pltpu.sync_copy(dst_hbm.at[pl.ds(wid * CPT, CPT)], dstbuf)
    for k in range(CHUNK // 16):
        ones_v[pl.ds(16 * k, 16)] = jnp.ones((16,), jnp.float32)
    if CHUNK % 16:
        ones_v[pl.ds(CHUNK - 16, 16)] = jnp.ones((16,), jnp.float32)
    pltpu.sync_copy(zeros_hbm.at[pl.ds(s * DPT, DPT)], deg_sh.at[pl.ds(s * DPT, DPT)])
    plsc.subcore_barrier()

    def body(j, carry):
        pltpu.sync_copy(ones_v, deg_sh.at[dstbuf.at[j]], add=True)
        return carry

    lax.fori_loop(0, CPT, body, 0)
    plsc.subcore_barrier()
    pltpu.sync_copy(deg_sh.at[pl.ds(s * DPT, DPT)], out_hbm.at[c, pl.ds(s * DPT, DPT)])


def _sc_degree(dst2, zeros1d):
    fn = pl.kernel(
        _sc_deg_body,
        out_type=jax.ShapeDtypeStruct((NC, NPAD), jnp.float32),
        mesh=_mesh(),
        scratch_types=[
            pltpu.VMEM((CPT, CHUNK), jnp.int32),
            pltpu.VMEM((CHUNK,), jnp.float32),
            pltpu.SemaphoreType.DMA,
            pltpu.VMEM_SHARED((NPAD,), jnp.float32),
        ],
    )
    return fn(dst2, zeros1d)


def _sc_edge_body(g_hbm, src_hbm, dst_hbm, zeros_hbm, out_hbm,
                  srcbuf, dstbuf, rowbuf_a, rowbuf_b, sem_a, sem_b, acc_sh):
    c = lax.axis_index("c")
    s = lax.axis_index("s")
    wid = c * NS + s
    node_base = s * NPT

    @pl.when(jnp.logical_and(c == 0, s < NS - 1))
    def _():
        pltpu.sync_copy(g_hbm.at[pl.ds(node_base, NPT)],
                        acc_sh.at[pl.ds(node_base, NPT)])

    @pl.when(jnp.logical_and(c == 0, s == NS - 1))
    def _():
        pltpu.sync_copy(g_hbm.at[pl.ds(node_base, NPT_LAST)],
                        acc_sh.at[pl.ds(node_base, NPT_LAST)])

    @pl.when(jnp.logical_and(c != 0, s < NS - 1))
    def _():
        pltpu.sync_copy(zeros_hbm.at[pl.ds(node_base, NPT)],
                        acc_sh.at[pl.ds(node_base, NPT)])

    @pl.when(jnp.logical_and(c != 0, s == NS - 1))
    def _():
        pltpu.sync_copy(zeros_hbm.at[pl.ds(node_base, NPT_LAST)],
                        acc_sh.at[pl.ds(node_base, NPT_LAST)])

    plsc.subcore_barrier()

    for half in range(CPT // HCPT):
        base = wid * CPT + half * HCPT
        pltpu.sync_copy(src_hbm.at[pl.ds(base, HCPT)], srcbuf)
        pltpu.sync_copy(dst_hbm.at[pl.ds(base, HCPT)], dstbuf)
        pltpu.async_copy(g_hbm.at[srcbuf.at[0]], rowbuf_a, sem_a)

        def body(p, carry):
            ja = 2 * p
            pltpu.async_copy(g_hbm.at[srcbuf.at[ja + 1]], rowbuf_b, sem_b)
            pltpu.make_async_copy(
                g_hbm.at[srcbuf.at[ja]], rowbuf_a, sem_a).wait()
            pltpu.sync_copy(rowbuf_a, acc_sh.at[dstbuf.at[ja]], add=True)

            @pl.when(ja + 2 < HCPT)
            def _():
                pltpu.async_copy(g_hbm.at[srcbuf.at[ja + 2]], rowbuf_a, sem_a)

            pltpu.make_async_copy(
                g_hbm.at[srcbuf.at[ja]], rowbuf_b, sem_b).wait()
            pltpu.sync_copy(rowbuf_b, acc_sh.at[dstbuf.at[ja + 1]], add=True)
            return carry

        lax.fori_loop(0, HCPT // 2, body, 0)
    plsc.subcore_barrier()

    @pl.when(s < NS - 1)
    def _():
        pltpu.sync_copy(acc_sh.at[pl.ds(node_base, NPT)],
                        out_hbm.at[c, pl.ds(node_base, NPT)])

    @pl.when(s == NS - 1)
    def _():
        pltpu.sync_copy(acc_sh.at[pl.ds(node_base, NPT_LAST)],
                        out_hbm.at[c, pl.ds(node_base, NPT_LAST)])


def _sc_edge(g, src2, dst2, zeros2d):
    fn = pl.kernel(
        _sc_edge_body,
        out_type=jax.ShapeDtypeStruct((NC, N, D), jnp.float32),
        mesh=_mesh(),
        scratch_types=[
            pltpu.VMEM((HCPT, CHUNK), jnp.int32),
            pltpu.VMEM((HCPT, CHUNK), jnp.int32),
            pltpu.VMEM((CHUNK, D), jnp.float32),
            pltpu.VMEM((CHUNK, D), jnp.float32),
            pltpu.SemaphoreType.DMA,
            pltpu.SemaphoreType.DMA,
            pltpu.VMEM_SHARED((N, D), jnp.float32),
        ],
    )
    return fn(g, src2, dst2, zeros2d)


def _tc_first_body(x_ref, w_ref, d0_ref, d1_ref, g_ref, dinv_ref):
    deg = d0_ref[0, 0, :] + d1_ref[0, 0, :] + 1.0
    dinv = lax.rsqrt(deg)
    dinv_ref[0, 0, :] = dinv
    xw = jnp.dot(x_ref[...], w_ref[...], preferred_element_type=jnp.float32)
    g_ref[...] = xw * dinv[:, None]


def _tc_mid_body(a_ref, dinv_ref, b_ref, w_ref, g_ref):
    dinv = dinv_ref[0, 0, :]
    acc = a_ref[0, :, :] + a_ref[1, :, :]
    h = jnp.maximum(acc * dinv[:, None] + b_ref[...], 0.0)
    hw = jnp.dot(h, w_ref[...], preferred_element_type=jnp.float32)
    g_ref[...] = hw * dinv[:, None]


def _tc_final_body(a_ref, dinv_ref, b_ref, batch_ref, w1_ref, b1_ref,
                   w2_ref, b2_ref, out_ref, psum, cnt):
    i = pl.program_id(0)

    @pl.when(i == 0)
    def _():
        psum[...] = jnp.zeros_like(psum)
        cnt[...] = jnp.zeros_like(cnt)

    dinv = dinv_ref[0, 0, :]
    h = jnp.maximum((a_ref[0, :, :] + a_ref[1, :, :]) * dinv[:, None]
                    + b_ref[...], 0.0)
    bt = batch_ref[0, 0, :]
    seg = lax.broadcasted_iota(jnp.int32, (G, BLK), 0)
    onehot = (bt[None, :] == seg).astype(jnp.float32)
    psum[...] += jnp.dot(onehot, h, preferred_element_type=jnp.float32)
    cnt[...] += jnp.broadcast_to(
        jnp.sum(onehot, axis=1, keepdims=True), cnt.shape)

    @pl.when(i == pl.num_programs(0) - 1)
    def _():
        pooled = psum[...] / jnp.maximum(cnt[...], 1.0)
        z = jnp.maximum(
            jnp.dot(pooled, w1_ref[...], preferred_element_type=jnp.float32)
            + b1_ref[...], 0.0)
        out_ref[...] = (jnp.sum(z * w2_ref[...], axis=1, keepdims=True)
                        + b2_ref[...])


def _tc_first(x, W0, deg0, deg1):
    return pl.pallas_call(
        _tc_first_body,
        grid=(NBLK,),
        in_specs=[
            pl.BlockSpec((BLK, D), lambda i: (i, 0)),
            pl.BlockSpec((D, D), lambda i: (0, 0)),
            pl.BlockSpec((1, 1, BLK), lambda i: (i, 0, 0)),
            pl.BlockSpec((1, 1, BLK), lambda i: (i, 0, 0)),
        ],
        out_specs=[
            pl.BlockSpec((BLK, D), lambda i: (i, 0)),
            pl.BlockSpec((1, 1, BLK), lambda i: (i, 0, 0)),
        ],
        out_shape=[
            jax.ShapeDtypeStruct((N, D), jnp.float32),
            jax.ShapeDtypeStruct((NBLK, 1, BLK), jnp.float32),
        ],
    )(x, W0, deg0, deg1)


def _tc_mid(acc, dinv3, b_row, W):
    return pl.pallas_call(
        _tc_mid_body,
        grid=(NBLK,),
        in_specs=[
            pl.BlockSpec((NC, BLK, D), lambda i: (0, i, 0)),
            pl.BlockSpec((1, 1, BLK), lambda i: (i, 0, 0)),
            pl.BlockSpec((1, D), lambda i: (0, 0)),
            pl.BlockSpec((D, D), lambda i: (0, 0)),
        ],
        out_specs=pl.BlockSpec((BLK, D), lambda i: (i, 0)),
        out_shape=jax.ShapeDtypeStruct((N, D), jnp.float32),
    )(acc, dinv3, b_row, W)


def _tc_final(acc, dinv3, b_row, batch3, W1, b1_row, w2_row, b2_11):
    return pl.pallas_call(
        _tc_final_body,
        grid=(NBLK,),
        in_specs=[
            pl.BlockSpec((NC, BLK, D), lambda i: (0, i, 0)),
            pl.BlockSpec((1, 1, BLK), lambda i: (i, 0, 0)),
            pl.BlockSpec((1, D), lambda i: (0, 0)),
            pl.BlockSpec((1, 1, BLK), lambda i: (i, 0, 0)),
            pl.BlockSpec((D, H), lambda i: (0, 0)),
            pl.BlockSpec((1, H), lambda i: (0, 0)),
            pl.BlockSpec((1, H), lambda i: (0, 0)),
            pl.BlockSpec((1, 1), lambda i: (0, 0)),
        ],
        out_specs=pl.BlockSpec((G, 1), lambda i: (0, 0)),
        out_shape=jax.ShapeDtypeStruct((G, 1), jnp.float32),
        scratch_shapes=[
            pltpu.VMEM((G, D), jnp.float32),
            pltpu.VMEM((G, D), jnp.float32),
        ],
    )(acc, dinv3, b_row, batch3, W1, b1_row, w2_row, b2_11)


def kernel(x, edge_index, batch, conv_W0, conv_b0, conv_W1, conv_b1,
           conv_W2, conv_b2, lin1_W, lin1_b, lin2_W, lin2_b):
    src2 = edge_index[0].reshape(E // CHUNK, CHUNK)
    dst2 = edge_index[1].reshape(E // CHUNK, CHUNK)
    zeros2d = jnp.zeros((N, D), jnp.float32)
    zeros1d = jnp.zeros((NPAD,), jnp.float32)
    batch3 = batch.reshape(NBLK, 1, BLK)

    deg_pair = _sc_degree(dst2, zeros1d)
    deg0 = deg_pair[0, :N].reshape(NBLK, 1, BLK)
    deg1 = deg_pair[1, :N].reshape(NBLK, 1, BLK)

    g, dinv3 = _tc_first(x, conv_W0, deg0, deg1)

    bs = [conv_b0, conv_b1, conv_b2]
    Ws = [conv_W1, conv_W2]
    for l in range(2):
        acc = _sc_edge(g, src2, dst2, zeros2d)
        g = _tc_mid(acc, dinv3, bs[l].reshape(1, D), Ws[l])
    acc = _sc_edge(g, src2, dst2, zeros2d)

    return _tc_final(acc, dinv3, bs[2].reshape(1, D), batch3,
                     lin1_W, lin1_b.reshape(1, H), lin2_W.reshape(1, H),
                     lin2_b.reshape(1, 1))


# speedup vs baseline: 30.0976x; 1.0316x over previous
import functools

import jax
import jax.numpy as jnp
from jax import lax
from jax.experimental import pallas as pl
from jax.experimental.pallas import tpu as pltpu
from jax.experimental.pallas import tpu_sc as plsc

N = 10000
D = 128
E = 320000
G = 64
H = 256

NC = 2
NS = 16
NW = NC * NS
CHUNK = 125
CPT = E // (NW * CHUNK)
EPT = E // NW
NPT = 624
NPT_LAST = N - (NS - 1) * NPT
HCPT = CPT // 2
ZROWS = 120
NPAD = 10240
DPT = NPAD // NS

BLK = 1000
NBLK = N // BLK


def _mesh():
    return plsc.VectorSubcoreMesh(core_axis_name="c", subcore_axis_name="s")


def _sc_deg_body(dst_hbm, zeros_hbm, out_hbm, dstbuf, ones_v, sem, deg_sh):
    c = lax.axis_index("c")
    s = lax.axis_index("s")
    wid = c * NS + s
    pltpu.sync_copy(dst_hbm.at[pl.ds(wid * CPT, CPT)], dstbuf)
    for k in range(CHUNK // 16):
        ones_v[pl.ds(16 * k, 16)] = jnp.ones((16,), jnp.float32)
    if CHUNK % 16:
        ones_v[pl.ds(CHUNK - 16, 16)] = jnp.ones((16,), jnp.float32)
    pltpu.sync_copy(zeros_hbm.at[pl.ds(s * DPT, DPT)], deg_sh.at[pl.ds(s * DPT, DPT)])
    plsc.subcore_barrier()

    def body(j, carry):
        pltpu.sync_copy(ones_v, deg_sh.at[dstbuf.at[j]], add=True)
        return carry

    lax.fori_loop(0, CPT, body, 0)
    plsc.subcore_barrier()
    pltpu.sync_copy(deg_sh.at[pl.ds(s * DPT, DPT)], out_hbm.at[c, pl.ds(s * DPT, DPT)])


def _sc_degree(dst2, zeros1d):
    fn = pl.kernel(
        _sc_deg_body,
        out_type=jax.ShapeDtypeStruct((NC, NPAD), jnp.float32),
        mesh=_mesh(),
        scratch_types=[
            pltpu.VMEM((CPT, CHUNK), jnp.int32),
            pltpu.VMEM((CHUNK,), jnp.float32),
            pltpu.SemaphoreType.DMA,
            pltpu.VMEM_SHARED((NPAD,), jnp.float32),
        ],
    )
    return fn(dst2, zeros1d)


def _sc_edge_body(g_hbm, src_hbm, dst_hbm, out_hbm,
                  srcbuf, dstbuf, rowbuf_a, rowbuf_b, sem_a, sem_b, acc_sh):
    c = lax.axis_index("c")
    s = lax.axis_index("s")
    wid = c * NS + s
    node_base = s * NPT

    pltpu.sync_copy(src_hbm.at[pl.ds(wid * CPT, HCPT)], srcbuf)
    pltpu.sync_copy(dst_hbm.at[pl.ds(wid * CPT, HCPT)], dstbuf)
    pltpu.async_copy(g_hbm.at[srcbuf.at[0]], rowbuf_a, sem_a)

    def zbody(i, carry):
        for k in range(D // 16):
            rowbuf_b[i, pl.ds(16 * k, 16)] = jnp.zeros((16,), jnp.float32)
        return carry

    lax.fori_loop(0, ZROWS, zbody, 0)
    for k in range(NPT // ZROWS):
        pltpu.sync_copy(rowbuf_b.at[pl.ds(0, ZROWS)],
                        acc_sh.at[pl.ds(node_base + k * ZROWS, ZROWS)])

    @pl.when(s < NS - 1)
    def _():
        pltpu.sync_copy(rowbuf_b.at[pl.ds(0, NPT % ZROWS)],
                        acc_sh.at[pl.ds(node_base + NPT - NPT % ZROWS,
                                        NPT % ZROWS)])

    @pl.when(s == NS - 1)
    def _():
        pltpu.sync_copy(rowbuf_b.at[pl.ds(0, NPT_LAST - NPT + NPT % ZROWS)],
                        acc_sh.at[pl.ds(node_base + NPT - NPT % ZROWS,
                                        NPT_LAST - NPT + NPT % ZROWS)])

    plsc.subcore_barrier()

    for half in range(CPT // HCPT):
        base = wid * CPT + half * HCPT
        if half > 0:
            pltpu.sync_copy(src_hbm.at[pl.ds(base, HCPT)], srcbuf)
            pltpu.sync_copy(dst_hbm.at[pl.ds(base, HCPT)], dstbuf)
            pltpu.async_copy(g_hbm.at[srcbuf.at[0]], rowbuf_a, sem_a)

        def body(p, carry):
            ja = 2 * p
            pltpu.async_copy(g_hbm.at[srcbuf.at[ja + 1]], rowbuf_b, sem_b)
            pltpu.make_async_copy(
                g_hbm.at[srcbuf.at[ja]], rowbuf_a, sem_a).wait()
            pltpu.sync_copy(rowbuf_a, acc_sh.at[dstbuf.at[ja]], add=True)

            @pl.when(ja + 2 < HCPT)
            def _():
                pltpu.async_copy(g_hbm.at[srcbuf.at[ja + 2]], rowbuf_a, sem_a)

            pltpu.make_async_copy(
                g_hbm.at[srcbuf.at[ja]], rowbuf_b, sem_b).wait()
            pltpu.sync_copy(rowbuf_b, acc_sh.at[dstbuf.at[ja + 1]], add=True)
            return carry

        lax.fori_loop(0, HCPT // 2, body, 0)
    plsc.subcore_barrier()

    @pl.when(s < NS - 1)
    def _():
        pltpu.sync_copy(acc_sh.at[pl.ds(node_base, NPT)],
                        out_hbm.at[c, pl.ds(node_base, NPT)])

    @pl.when(s == NS - 1)
    def _():
        pltpu.sync_copy(acc_sh.at[pl.ds(node_base, NPT_LAST)],
                        out_hbm.at[c, pl.ds(node_base, NPT_LAST)])


def _sc_edge(g, src2, dst2):
    fn = pl.kernel(
        _sc_edge_body,
        out_type=jax.ShapeDtypeStruct((NC, N, D), jnp.float32),
        mesh=_mesh(),
        scratch_types=[
            pltpu.VMEM((HCPT, CHUNK), jnp.int32),
            pltpu.VMEM((HCPT, CHUNK), jnp.int32),
            pltpu.VMEM((CHUNK, D), jnp.float32),
            pltpu.VMEM((CHUNK, D), jnp.float32),
            pltpu.SemaphoreType.DMA,
            pltpu.SemaphoreType.DMA,
            pltpu.VMEM_SHARED((N, D), jnp.float32),
        ],
    )
    return fn(g, src2, dst2)


def _tc_first_body(x_ref, w_ref, d0_ref, d1_ref, g_ref, dinv_ref):
    deg = d0_ref[0, 0, :] + d1_ref[0, 0, :] + 1.0
    dinv = lax.rsqrt(deg)
    dinv_ref[0, 0, :] = dinv
    xw = jnp.dot(x_ref[...], w_ref[...], preferred_element_type=jnp.float32)
    g_ref[...] = xw * dinv[:, None]


def _tc_mid_body(a_ref, gp_ref, dinv_ref, b_ref, w_ref, g_ref):
    dinv = dinv_ref[0, 0, :]
    acc = a_ref[0, :, :] + a_ref[1, :, :] + gp_ref[...]
    h = jnp.maximum(acc * dinv[:, None] + b_ref[...], 0.0)
    hw = jnp.dot(h, w_ref[...], preferred_element_type=jnp.float32)
    g_ref[...] = hw * dinv[:, None]


def _tc_final_body(a_ref, gp_ref, dinv_ref, b_ref, batch_ref, w1_ref, b1_ref,
                   w2_ref, b2_ref, out_ref, psum, cnt):
    i = pl.program_id(0)

    @pl.when(i == 0)
    def _():
        psum[...] = jnp.zeros_like(psum)
        cnt[...] = jnp.zeros_like(cnt)

    dinv = dinv_ref[0, 0, :]
    h = jnp.maximum(
        (a_ref[0, :, :] + a_ref[1, :, :] + gp_ref[...]) * dinv[:, None]
        + b_ref[...], 0.0)
    bt = batch_ref[0, 0, :]
    seg = lax.broadcasted_iota(jnp.int32, (G, BLK), 0)
    onehot = (bt[None, :] == seg).astype(jnp.float32)
    psum[...] += jnp.dot(onehot, h, preferred_element_type=jnp.float32)
    cnt[...] += jnp.broadcast_to(
        jnp.sum(onehot, axis=1, keepdims=True), cnt.shape)

    @pl.when(i == pl.num_programs(0) - 1)
    def _():
        pooled = psum[...] / jnp.maximum(cnt[...], 1.0)
        z = jnp.maximum(
            jnp.dot(pooled, w1_ref[...], preferred_element_type=jnp.float32)
            + b1_ref[...], 0.0)
        out_ref[...] = (jnp.sum(z * w2_ref[...], axis=1, keepdims=True)
                        + b2_ref[...])


def _tc_first(x, W0, deg0, deg1):
    return pl.pallas_call(
        _tc_first_body,
        grid=(NBLK,),
        in_specs=[
            pl.BlockSpec((BLK, D), lambda i: (i, 0)),
            pl.BlockSpec((D, D), lambda i: (0, 0)),
            pl.BlockSpec((1, 1, BLK), lambda i: (i, 0, 0)),
            pl.BlockSpec((1, 1, BLK), lambda i: (i, 0, 0)),
        ],
        out_specs=[
            pl.BlockSpec((BLK, D), lambda i: (i, 0)),
            pl.BlockSpec((1, 1, BLK), lambda i: (i, 0, 0)),
        ],
        out_shape=[
            jax.ShapeDtypeStruct((N, D), jnp.float32),
            jax.ShapeDtypeStruct((NBLK, 1, BLK), jnp.float32),
        ],
    )(x, W0, deg0, deg1)


def _tc_mid(acc, g_prev, dinv3, b_row, W):
    return pl.pallas_call(
        _tc_mid_body,
        grid=(NBLK,),
        in_specs=[
            pl.BlockSpec((NC, BLK, D), lambda i: (0, i, 0)),
            pl.BlockSpec((BLK, D), lambda i: (i, 0)),
            pl.BlockSpec((1, 1, BLK), lambda i: (i, 0, 0)),
            pl.BlockSpec((1, D), lambda i: (0, 0)),
            pl.BlockSpec((D, D), lambda i: (0, 0)),
        ],
        out_specs=pl.BlockSpec((BLK, D), lambda i: (i, 0)),
        out_shape=jax.ShapeDtypeStruct((N, D), jnp.float32),
    )(acc, g_prev, dinv3, b_row, W)


def _tc_final(acc, g_prev, dinv3, b_row, batch3, W1, b1_row, w2_row, b2_11):
    return pl.pallas_call(
        _tc_final_body,
        grid=(NBLK,),
        in_specs=[
            pl.BlockSpec((NC, BLK, D), lambda i: (0, i, 0)),
            pl.BlockSpec((BLK, D), lambda i: (i, 0)),
            pl.BlockSpec((1, 1, BLK), lambda i: (i, 0, 0)),
            pl.BlockSpec((1, D), lambda i: (0, 0)),
            pl.BlockSpec((1, 1, BLK), lambda i: (i, 0, 0)),
            pl.BlockSpec((D, H), lambda i: (0, 0)),
            pl.BlockSpec((1, H), lambda i: (0, 0)),
            pl.BlockSpec((1, H), lambda i: (0, 0)),
            pl.BlockSpec((1, 1), lambda i: (0, 0)),
        ],
        out_specs=pl.BlockSpec((G, 1), lambda i: (0, 0)),
        out_shape=jax.ShapeDtypeStruct((G, 1), jnp.float32),
        scratch_shapes=[
            pltpu.VMEM((G, D), jnp.float32),
            pltpu.VMEM((G, D), jnp.float32),
        ],
    )(acc, g_prev, dinv3, b_row, batch3, W1, b1_row, w2_row, b2_11)


def kernel(x, edge_index, batch, conv_W0, conv_b0, conv_W1, conv_b1,
           conv_W2, conv_b2, lin1_W, lin1_b, lin2_W, lin2_b):
    src2 = edge_index[0].reshape(E // CHUNK, CHUNK)
    dst2 = edge_index[1].reshape(E // CHUNK, CHUNK)
    zeros1d = jnp.zeros((NPAD,), jnp.float32)
    batch3 = batch.reshape(NBLK, 1, BLK)

    deg_pair = _sc_degree(dst2, zeros1d)
    deg0 = deg_pair[0, :N].reshape(NBLK, 1, BLK)
    deg1 = deg_pair[1, :N].reshape(NBLK, 1, BLK)

    g, dinv3 = _tc_first(x, conv_W0, deg0, deg1)

    bs = [conv_b0, conv_b1, conv_b2]
    Ws = [conv_W1, conv_W2]
    for l in range(2):
        acc = _sc_edge(g, src2, dst2)
        g = _tc_mid(acc, g, dinv3, bs[l].reshape(1, D), Ws[l])
    acc = _sc_edge(g, src2, dst2)

    return _tc_final(acc, g, dinv3, bs[2].reshape(1, D), batch3,
                     lin1_W, lin1_b.reshape(1, H), lin2_W.reshape(1, H),
                     lin2_b.reshape(1, 1))


# speedup vs baseline: 30.1297x; 1.0011x over previous
import functools

import jax
import jax.numpy as jnp
from jax import lax
from jax.experimental import pallas as pl
from jax.experimental.pallas import tpu as pltpu
from jax.experimental.pallas import tpu_sc as plsc

N = 10000
D = 128
E = 320000
G = 64
H = 256

NC = 2
NS = 16
NW = NC * NS
CHUNK = 125
CPT = E // (NW * CHUNK)
EPT = E // NW
NPT = 624
NPT_LAST = N - (NS - 1) * NPT
HCPT = CPT // 2
ZROWS = 120
GA = 64
GB = CHUNK - GA
NPAD = 10240
DPT = NPAD // NS

BLK = 1000
NBLK = N // BLK


def _mesh():
    return plsc.VectorSubcoreMesh(core_axis_name="c", subcore_axis_name="s")


def _sc_deg_body(dst_hbm, zeros_hbm, out_hbm, dstbuf, ones_v, sem, deg_sh):
    c = lax.axis_index("c")
    s = lax.axis_index("s")
    wid = c * NS + s
    pltpu.sync_copy(dst_hbm.at[pl.ds(wid * CPT, CPT)], dstbuf)
    for k in range(CHUNK // 16):
        ones_v[pl.ds(16 * k, 16)] = jnp.ones((16,), jnp.float32)
    if CHUNK % 16:
        ones_v[pl.ds(CHUNK - 16, 16)] = jnp.ones((16,), jnp.float32)
    pltpu.sync_copy(zeros_hbm.at[pl.ds(s * DPT, DPT)], deg_sh.at[pl.ds(s * DPT, DPT)])
    plsc.subcore_barrier()

    def body(j, carry):
        pltpu.sync_copy(ones_v, deg_sh.at[dstbuf.at[j]], add=True)
        return carry

    lax.fori_loop(0, CPT, body, 0)
    plsc.subcore_barrier()
    pltpu.sync_copy(deg_sh.at[pl.ds(s * DPT, DPT)], out_hbm.at[c, pl.ds(s * DPT, DPT)])


def _sc_degree(dst2, zeros1d):
    fn = pl.kernel(
        _sc_deg_body,
        out_type=jax.ShapeDtypeStruct((NC, NPAD), jnp.float32),
        mesh=_mesh(),
        scratch_types=[
            pltpu.VMEM((CPT, CHUNK), jnp.int32),
            pltpu.VMEM((CHUNK,), jnp.float32),
            pltpu.SemaphoreType.DMA,
            pltpu.VMEM_SHARED((NPAD,), jnp.float32),
        ],
    )
    return fn(dst2, zeros1d)


def _sc_edge_body(g_hbm, src_hbm, dst_hbm, out_hbm,
                  srcbuf, dstbuf, rowbuf_a, rowbuf_b, sem_a, sem_b, acc_sh):
    c = lax.axis_index("c")
    s = lax.axis_index("s")
    wid = c * NS + s
    node_base = s * NPT

    def gather(j, buf, sem):
        pltpu.async_copy(g_hbm.at[srcbuf.at[j, pl.ds(0, GA)]],
                         buf.at[pl.ds(0, GA)], sem)
        pltpu.async_copy(g_hbm.at[srcbuf.at[j, pl.ds(GA, GB)]],
                         buf.at[pl.ds(GA, GB)], sem)

    def gather_wait(buf, sem):
        pltpu.make_async_copy(g_hbm.at[srcbuf.at[0, pl.ds(0, GA)]],
                              buf.at[pl.ds(0, GA)], sem).wait()
        pltpu.make_async_copy(g_hbm.at[srcbuf.at[0, pl.ds(GA, GB)]],
                              buf.at[pl.ds(GA, GB)], sem).wait()

    pltpu.sync_copy(src_hbm.at[pl.ds(wid * CPT, HCPT)], srcbuf)
    pltpu.sync_copy(dst_hbm.at[pl.ds(wid * CPT, HCPT)], dstbuf)
    gather(0, rowbuf_a, sem_a)

    def zbody(i, carry):
        for k in range(D // 16):
            rowbuf_b[i, pl.ds(16 * k, 16)] = jnp.zeros((16,), jnp.float32)
        return carry

    lax.fori_loop(0, ZROWS, zbody, 0)
    for k in range(NPT // ZROWS):
        pltpu.sync_copy(rowbuf_b.at[pl.ds(0, ZROWS)],
                        acc_sh.at[pl.ds(node_base + k * ZROWS, ZROWS)])

    @pl.when(s < NS - 1)
    def _():
        pltpu.sync_copy(rowbuf_b.at[pl.ds(0, NPT % ZROWS)],
                        acc_sh.at[pl.ds(node_base + NPT - NPT % ZROWS,
                                        NPT % ZROWS)])

    @pl.when(s == NS - 1)
    def _():
        pltpu.sync_copy(rowbuf_b.at[pl.ds(0, NPT_LAST - NPT + NPT % ZROWS)],
                        acc_sh.at[pl.ds(node_base + NPT - NPT % ZROWS,
                                        NPT_LAST - NPT + NPT % ZROWS)])

    plsc.subcore_barrier()

    for half in range(CPT // HCPT):
        base = wid * CPT + half * HCPT
        if half > 0:
            pltpu.sync_copy(src_hbm.at[pl.ds(base, HCPT)], srcbuf)
            pltpu.sync_copy(dst_hbm.at[pl.ds(base, HCPT)], dstbuf)
            gather(0, rowbuf_a, sem_a)

        def body(p, carry):
            ja = 2 * p
            gather(ja + 1, rowbuf_b, sem_b)
            gather_wait(rowbuf_a, sem_a)
            pltpu.sync_copy(rowbuf_a, acc_sh.at[dstbuf.at[ja]], add=True)

            @pl.when(ja + 2 < HCPT)
            def _():
                gather(ja + 2, rowbuf_a, sem_a)

            gather_wait(rowbuf_b, sem_b)
            pltpu.sync_copy(rowbuf_b, acc_sh.at[dstbuf.at[ja + 1]], add=True)
            return carry

        lax.fori_loop(0, HCPT // 2, body, 0)
    plsc.subcore_barrier()

    @pl.when(s < NS - 1)
    def _():
        pltpu.sync_copy(acc_sh.at[pl.ds(node_base, NPT)],
                        out_hbm.at[c, pl.ds(node_base, NPT)])

    @pl.when(s == NS - 1)
    def _():
        pltpu.sync_copy(acc_sh.at[pl.ds(node_base, NPT_LAST)],
                        out_hbm.at[c, pl.ds(node_base, NPT_LAST)])


def _sc_edge(g, src2, dst2):
    fn = pl.kernel(
        _sc_edge_body,
        out_type=jax.ShapeDtypeStruct((NC, N, D), jnp.float32),
        mesh=_mesh(),
        scratch_types=[
            pltpu.VMEM((HCPT, CHUNK), jnp.int32),
            pltpu.VMEM((HCPT, CHUNK), jnp.int32),
            pltpu.VMEM((CHUNK, D), jnp.float32),
            pltpu.VMEM((CHUNK, D), jnp.float32),
            pltpu.SemaphoreType.DMA,
            pltpu.SemaphoreType.DMA,
            pltpu.VMEM_SHARED((N, D), jnp.float32),
        ],
    )
    return fn(g, src2, dst2)


def _tc_first_body(x_ref, w_ref, d0_ref, d1_ref, g_ref, dinv_ref):
    deg = d0_ref[0, 0, :] + d1_ref[0, 0, :] + 1.0
    dinv = lax.rsqrt(deg)
    dinv_ref[0, 0, :] = dinv
    xw = jnp.dot(x_ref[...], w_ref[...], preferred_element_type=jnp.float32)
    g_ref[...] = xw * dinv[:, None]


def _tc_mid_body(a_ref, gp_ref, dinv_ref, b_ref, w_ref, g_ref):
    dinv = dinv_ref[0, 0, :]
    acc = a_ref[0, :, :] + a_ref[1, :, :] + gp_ref[...]
    h = jnp.maximum(acc * dinv[:, None] + b_ref[...], 0.0)
    hw = jnp.dot(h, w_ref[...], preferred_element_type=jnp.float32)
    g_ref[...] = hw * dinv[:, None]


def _tc_final_body(a_ref, gp_ref, dinv_ref, b_ref, batch_ref, w1_ref, b1_ref,
                   w2_ref, b2_ref, out_ref, psum, cnt):
    i = pl.program_id(0)

    @pl.when(i == 0)
    def _():
        psum[...] = jnp.zeros_like(psum)
        cnt[...] = jnp.zeros_like(cnt)

    dinv = dinv_ref[0, 0, :]
    h = jnp.maximum(
        (a_ref[0, :, :] + a_ref[1, :, :] + gp_ref[...]) * dinv[:, None]
        + b_ref[...], 0.0)
    bt = batch_ref[0, 0, :]
    seg = lax.broadcasted_iota(jnp.int32, (G, BLK), 0)
    onehot = (bt[None, :] == seg).astype(jnp.float32)
    psum[...] += jnp.dot(onehot, h, preferred_element_type=jnp.float32)
    cnt[...] += jnp.broadcast_to(
        jnp.sum(onehot, axis=1, keepdims=True), cnt.shape)

    @pl.when(i == pl.num_programs(0) - 1)
    def _():
        pooled = psum[...] / jnp.maximum(cnt[...], 1.0)
        z = jnp.maximum(
            jnp.dot(pooled, w1_ref[...], preferred_element_type=jnp.float32)
            + b1_ref[...], 0.0)
        out_ref[...] = (jnp.sum(z * w2_ref[...], axis=1, keepdims=True)
                        + b2_ref[...])


def _tc_first(x, W0, deg0, deg1):
    return pl.pallas_call(
        _tc_first_body,
        grid=(NBLK,),
        in_specs=[
            pl.BlockSpec((BLK, D), lambda i: (i, 0)),
            pl.BlockSpec((D, D), lambda i: (0, 0)),
            pl.BlockSpec((1, 1, BLK), lambda i: (i, 0, 0)),
            pl.BlockSpec((1, 1, BLK), lambda i: (i, 0, 0)),
        ],
        out_specs=[
            pl.BlockSpec((BLK, D), lambda i: (i, 0)),
            pl.BlockSpec((1, 1, BLK), lambda i: (i, 0, 0)),
        ],
        out_shape=[
            jax.ShapeDtypeStruct((N, D), jnp.float32),
            jax.ShapeDtypeStruct((NBLK, 1, BLK), jnp.float32),
        ],
    )(x, W0, deg0, deg1)


def _tc_mid(acc, g_prev, dinv3, b_row, W):
    return pl.pallas_call(
        _tc_mid_body,
        grid=(NBLK,),
        in_specs=[
            pl.BlockSpec((NC, BLK, D), lambda i: (0, i, 0)),
            pl.BlockSpec((BLK, D), lambda i: (i, 0)),
            pl.BlockSpec((1, 1, BLK), lambda i: (i, 0, 0)),
            pl.BlockSpec((1, D), lambda i: (0, 0)),
            pl.BlockSpec((D, D), lambda i: (0, 0)),
        ],
        out_specs=pl.BlockSpec((BLK, D), lambda i: (i, 0)),
        out_shape=jax.ShapeDtypeStruct((N, D), jnp.float32),
    )(acc, g_prev, dinv3, b_row, W)


def _tc_final(acc, g_prev, dinv3, b_row, batch3, W1, b1_row, w2_row, b2_11):
    return pl.pallas_call(
        _tc_final_body,
        grid=(NBLK,),
        in_specs=[
            pl.BlockSpec((NC, BLK, D), lambda i: (0, i, 0)),
            pl.BlockSpec((BLK, D), lambda i: (i, 0)),
            pl.BlockSpec((1, 1, BLK), lambda i: (i, 0, 0)),
            pl.BlockSpec((1, D), lambda i: (0, 0)),
            pl.BlockSpec((1, 1, BLK), lambda i: (i, 0, 0)),
            pl.BlockSpec((D, H), lambda i: (0, 0)),
            pl.BlockSpec((1, H), lambda i: (0, 0)),
            pl.BlockSpec((1, H), lambda i: (0, 0)),
            pl.BlockSpec((1, 1), lambda i: (0, 0)),
        ],
        out_specs=pl.BlockSpec((G, 1), lambda i: (0, 0)),
        out_shape=jax.ShapeDtypeStruct((G, 1), jnp.float32),
        scratch_shapes=[
            pltpu.VMEM((G, D), jnp.float32),
            pltpu.VMEM((G, D), jnp.float32),
        ],
    )(acc, g_prev, dinv3, b_row, batch3, W1, b1_row, w2_row, b2_11)


def kernel(x, edge_index, batch, conv_W0, conv_b0, conv_W1, conv_b1,
           conv_W2, conv_b2, lin1_W, lin1_b, lin2_W, lin2_b):
    src2 = edge_index[0].reshape(E // CHUNK, CHUNK)
    dst2 = edge_index[1].reshape(E // CHUNK, CHUNK)
    zeros1d = jnp.zeros((NPAD,), jnp.float32)
    batch3 = batch.reshape(NBLK, 1, BLK)

    deg_pair = _sc_degree(dst2, zeros1d)
    deg0 = deg_pair[0, :N].reshape(NBLK, 1, BLK)
    deg1 = deg_pair[1, :N].reshape(NBLK, 1, BLK)

    g, dinv3 = _tc_first(x, conv_W0, deg0, deg1)

    bs = [conv_b0, conv_b1, conv_b2]
    Ws = [conv_W1, conv_W2]
    for l in range(2):
        acc = _sc_edge(g, src2, dst2)
        g = _tc_mid(acc, g, dinv3, bs[l].reshape(1, D), Ws[l])
    acc = _sc_edge(g, src2, dst2)

    return _tc_final(acc, g, dinv3, bs[2].reshape(1, D), batch3,
                     lin1_W, lin1_b.reshape(1, H), lin2_W.reshape(1, H),
                     lin2_b.reshape(1, 1))
